# baseline (speedup 1.0000x reference)
import functools

import jax
import jax.numpy as jnp
from jax import lax
from jax.experimental import pallas as pl
from jax.experimental.pallas import tpu as pltpu
from jax.experimental.pallas import tpu_sc as plsc

_TOP_P = 0.9
_TOP_K = 64

_R = 128
_V = 100000
_NW = 32
_ROWS_PER_W = _R // _NW
_CHUNKS = _V // 16

_NBINS = 256
_HIST_SLOTS = 16 * _NBINS

_WCHUNKS = 10
_NWIN = _CHUNKS // _WCHUNKS

_HI_REGION = 96
_CAP_IN = 240
_BUF = _HI_REGION + _CAP_IN + 16
_NVREG = _BUF // 16

_BIG = 1 << 30


def _sc_body(logits_hbm, gumbel_hbm, kinf_hbm, tok_hbm, probs_hbm,
             row_v, hist, hist2, wmax, buf_v, buf_i, out_v, out_i, g_v,
             kinf_v, probs_st, tok_st, dma_sem):
    wid = lax.axis_index("s") * 2 + lax.axis_index("c")
    iota = lax.iota(jnp.int32, 16)
    lane0 = iota == 0
    zeros_i = jnp.zeros((16,), jnp.int32)
    ones_i = jnp.ones((16,), jnp.int32)
    neginf = jnp.full((16,), -jnp.inf, jnp.float32)
    lane_base = iota * jnp.int32(_NBINS)

    pltpu.sync_copy(kinf_hbm, kinf_v)
    first_row = wid * _ROWS_PER_W
    pltpu.async_copy(logits_hbm.at[first_row], row_v, dma_sem)

    def row_body(r, _):
        row = wid * _ROWS_PER_W + r
        pltpu.sync_copy(gumbel_hbm.at[row], g_v)

        def zh(z, _c):
            for u in range(8):
                hist[pl.ds((z * 8 + u) * 16, 16)] = zeros_i
                hist2[pl.ds((z * 8 + u) * 16, 16)] = zeros_i
            return 0
        lax.fori_loop(0, _HIST_SLOTS // 128, zh, 0)
        for z in range(_NVREG):
            buf_v[pl.ds(z * 16, 16)] = neginf

        pltpu.make_async_copy(logits_hbm.at[row], row_v, dma_sem).wait()

        def mm_body(c, carry):
            nlo_v, hi_v = carry
            for u in range(4):
                v = row_v[pl.ds((c * 4 + u) * 128, 16)]
                nlo_v = jnp.maximum(nlo_v, -v)
                hi_v = jnp.maximum(hi_v, v)
            return (nlo_v, hi_v)
        nlo_v, hi_v = lax.fori_loop(0, _CHUNKS // 32, mm_body,
                                    (neginf, neginf))
        lo = -jnp.max(nlo_v)
        scale = (jnp.full((16,), _NBINS - 2, jnp.float32)
                 / jnp.broadcast_to(jnp.max(hi_v) - lo, (16,)))

        def to_bin(v):
            ti = ((v - lo) * scale).astype(jnp.int32)
            return jnp.clip(ti, 0, _NBINS - 1)

        def hist_body(w, _c):
            wm = neginf
            for u in range(_WCHUNKS):
                v = row_v[pl.ds((w * _WCHUNKS + u) * 16, 16)]
                h = hist if u % 2 == 0 else hist2
                plsc.addupdate_scatter(h, [lane_base + to_bin(v)], ones_i)
                wm = jnp.maximum(wm, v)
            wmax[pl.ds(w * 16, 16)] = wm
            return 0
        lax.fori_loop(0, _NWIN, hist_body, 0)

        def scan_body(i, carry):
            found, bin_b, c_gt, n_b, cum = carry
            cc = (_NBINS // 16 - 1) - i

            chunk = zeros_i
            for sh in range(16):
                chunk = (chunk + hist[pl.ds(sh * _NBINS + cc * 16, 16)]
                         + hist2[pl.ds(sh * _NBINS + cc * 16, 16)])
            rchunk = lax.rev(chunk, (0,))
            sfx = plsc.cumsum(rchunk) + cum
            ge = sfx >= _TOP_K
            cnt_ge = jnp.sum(ge.astype(jnp.int32))
            pos = jnp.int32(16) - cnt_ge
            s_pos = jnp.min(jnp.where(ge, sfx, jnp.int32(_BIG)))
            prev = jnp.maximum(jnp.max(jnp.where(ge, jnp.int32(-_BIG), sfx)),
                               cum)
            hit = jnp.logical_and(jnp.logical_not(found), cnt_ge > 0)
            return (jnp.logical_or(found, cnt_ge > 0),
                    jnp.where(hit, cc * 16 + 15 - pos, bin_b),
                    jnp.where(hit, prev, c_gt),
                    jnp.where(hit, s_pos - prev, n_b),
                    cum + jnp.sum(rchunk))
        _, bin_b, c_gt, n_b, _ = lax.fori_loop(
            0, _NBINS // 16, scan_body,
            (jnp.bool_(False), jnp.int32(0), jnp.int32(0), jnp.int32(0),
             jnp.int32(0)))

        lf_v = (lo + (jnp.broadcast_to(bin_b, (16,)).astype(jnp.float32)
                      - 2.0) / scale)

        def col_window(w, carry):
            hit = jnp.any(wmax[pl.ds(w * 16, 16)] >= lf_v)

            def do(carry):
                cnt_hi, cnt_in = carry
                for u in range(_WCHUNKS):
                    c = w * _WCHUNKS + u
                    v = row_v[pl.ds(c * 16, 16)]
                    binv = to_bin(v)
                    mhi = binv > bin_b
                    min_ = jnp.logical_and(binv == bin_b, cnt_in < _CAP_IN)
                    nhi = jnp.sum(mhi.astype(jnp.int32))
                    nin = jnp.sum(min_.astype(jnp.int32))

                    @pl.when(nhi + nin > 0)
                    def _store(cnt_hi=cnt_hi, cnt_in=cnt_in, v=v,
                               mhi=mhi, min_=min_, c=c):
                        idxv = c * 16 + iota
                        plsc.store_compressed(buf_v.at[pl.ds(cnt_hi, 16)],
                                              v, mask=mhi)
                        plsc.store_compressed(buf_i.at[pl.ds(cnt_hi, 16)],
                                              idxv, mask=mhi)
                        plsc.store_compressed(
                            buf_v.at[pl.ds(_HI_REGION + cnt_in, 16)],
                            v, mask=min_)
                        plsc.store_compressed(
                            buf_i.at[pl.ds(_HI_REGION + cnt_in, 16)],
                            idxv, mask=min_)
                    cnt_hi = cnt_hi + nhi
                    cnt_in = cnt_in + nin
                return (cnt_hi, cnt_in)

            return lax.cond(hit, do, lambda c_: c_, carry)
        lax.fori_loop(0, _NWIN, col_window, (jnp.int32(0), jnp.int32(0)))

        @pl.when(r < _ROWS_PER_W - 1)
        def _prefetch():
            pltpu.async_copy(logits_hbm.at[row + 1], row_v, dma_sem)

        def sel_body(i, _c):
            bv, bp = neginf, jnp.full((16,), _BIG, jnp.int32)
            for j in range(_NVREG):
                x = buf_v[pl.ds(j * 16, 16)]
                m = x > bv
                bv = jnp.where(m, x, bv)
                bp = jnp.where(m, j * 16 + iota, bp)
            mx = jnp.max(bv)
            p = jnp.min(jnp.where(bv == mx, bp, jnp.int32(_BIG)))
            pv = jnp.broadcast_to(p, (16,))
            idx_p = jnp.max(plsc.load_gather(buf_i, [pv]))
            plsc.store_scatter(buf_v, [pv], neginf, mask=lane0)
            iv = jnp.broadcast_to(i, (16,))
            plsc.store_scatter(out_v, [iv], jnp.broadcast_to(mx, (16,)),
                               mask=lane0)
            plsc.store_scatter(out_i, [iv], jnp.broadcast_to(idx_p, (16,)),
                               mask=lane0)
            return 0
        lax.fori_loop(0, _TOP_K, sel_body, 0)

        vals = [out_v[pl.ds(j * 16, 16)] for j in range(4)]
        vmax = jnp.max(vals[0])
        es = [jnp.exp(v - vmax) for v in vals]
        s = es[0] + es[1] + es[2] + es[3]
        total = jnp.sum(s)
        keeps, masked = [], []
        carry = jnp.float32(0.0)
        for j in range(4):
            pj = es[j] / total
            cj = plsc.cumsum(pj) + carry
            carry = jnp.max(cj)
            kp = cj <= _TOP_P
            if j == 0:
                kp = jnp.logical_or(kp, lane0)
            keeps.append(kp)
            masked.append(jnp.where(kp, vals[j], -jnp.inf)
                          + kinf_v[pl.ds(j * 16, 16)])
        bm, bp = neginf, jnp.full((16,), _BIG, jnp.int32)
        for j in range(4):
            sc = masked[j] + g_v[pl.ds(j * 16, 16)]
            m = sc > bm
            bm = jnp.where(m, sc, bm)
            bp = jnp.where(m, j * 16 + iota, bp)
        mx2 = jnp.max(bm)
        p2 = jnp.min(jnp.where(bm == mx2, bp, jnp.int32(_BIG)))
        token = jnp.max(plsc.load_gather(out_i, [jnp.broadcast_to(p2, (16,))]))

        e2 = [jnp.where(jnp.logical_and(
                  keeps[j], kinf_v[pl.ds(j * 16, 16)] == 0.0),
                  es[j], 0.0) for j in range(4)]
        s2 = jnp.sum(e2[0] + e2[1] + e2[2] + e2[3])
        for j in range(4):
            probs_st[pl.ds(j * 16, 16)] = e2[j] / s2
        tok_st[...] = jnp.where(lane0, token, 0)

        pltpu.sync_copy(probs_st, probs_hbm.at[row])
        pltpu.sync_copy(tok_st, tok_hbm.at[row])
        return 0

    lax.fori_loop(0, _ROWS_PER_W, row_body, 0)


@functools.partial(jax.jit, static_argnames=())
def _sc_topk_sample(logits, gumbel, kinf):
    mesh = plsc.VectorSubcoreMesh(core_axis_name="c", subcore_axis_name="s")
    f = pl.kernel(
        _sc_body,
        mesh=mesh,
        compiler_params=pltpu.CompilerParams(needs_layout_passes=False),
        out_type=(
            jax.ShapeDtypeStruct((_R, 16), jnp.int32),
            jax.ShapeDtypeStruct((_R, _TOP_K), jnp.float32),
        ),
        scratch_types=[
            pltpu.VMEM((_V,), jnp.float32),
            pltpu.VMEM((_HIST_SLOTS,), jnp.int32),
            pltpu.VMEM((_HIST_SLOTS,), jnp.int32),
            pltpu.VMEM((_NWIN * 16,), jnp.float32),
            pltpu.VMEM((_BUF,), jnp.float32),
            pltpu.VMEM((_BUF,), jnp.int32),
            pltpu.VMEM((_TOP_K,), jnp.float32),
            pltpu.VMEM((_TOP_K,), jnp.int32),
            pltpu.VMEM((_TOP_K,), jnp.float32),
            pltpu.VMEM((_TOP_K,), jnp.float32),
            pltpu.VMEM((_TOP_K,), jnp.float32),
            pltpu.VMEM((16,), jnp.int32),
            pltpu.SemaphoreType.DMA,
        ],
    )
    return f(logits, gumbel, kinf)


def kernel(logits, k):
    gumbel = jax.random.gumbel(jax.random.key(42), (_R, _TOP_K), jnp.float32)
    kinf = jnp.where(jnp.arange(_TOP_K) < k, 0.0, -jnp.inf).astype(jnp.float32)
    tok, probs = _sc_topk_sample(logits, gumbel, kinf)
    return tok[:, 0], probs


# speedup vs baseline: 1.8630x; 1.8630x over previous
import functools

import jax
import jax.numpy as jnp
from jax import lax
from jax.experimental import pallas as pl
from jax.experimental.pallas import tpu as pltpu
from jax.experimental.pallas import tpu_sc as plsc

_TOP_P = 0.9
_TOP_K = 64

_R = 128
_V = 100000
_NW = 32
_ROWS_PER_W = _R // _NW
_CHUNKS = _V // 16

_NBINS = 256
_HIST_SLOTS = 16 * _NBINS

_WCHUNKS = 10
_NWIN = _CHUNKS // _WCHUNKS

_HI_REGION = 96
_CAP_IN = 240
_BUF = _HI_REGION + _CAP_IN + 16
_NVREG = _BUF // 16

_BIG = 1 << 30


def _sc_body(logits_hbm, gumbel_hbm, kinf_hbm, tok_hbm, probs_hbm,
             row_v, hist, hist2, wmax, buf_v, buf_i, out_v, out_i, g_v,
             kinf_v, probs_st, tok_st, dma_sem):
    wid = lax.axis_index("s") * 2 + lax.axis_index("c")
    iota = lax.iota(jnp.int32, 16)
    lane0 = iota == 0
    zeros_i = jnp.zeros((16,), jnp.int32)
    ones_i = jnp.ones((16,), jnp.int32)
    neginf = jnp.full((16,), -jnp.inf, jnp.float32)
    lane_base = iota * jnp.int32(_NBINS)

    pltpu.sync_copy(kinf_hbm, kinf_v)
    first_row = wid * _ROWS_PER_W
    pltpu.async_copy(logits_hbm.at[first_row], row_v, dma_sem)

    def row_body(r, _):
        row = wid * _ROWS_PER_W + r
        pltpu.sync_copy(gumbel_hbm.at[row], g_v)

        def zh(z, _c):
            for u in range(8):
                hist[pl.ds((z * 8 + u) * 16, 16)] = zeros_i
                hist2[pl.ds((z * 8 + u) * 16, 16)] = zeros_i
            return 0
        lax.fori_loop(0, _HIST_SLOTS // 128, zh, 0)
        for z in range(_NVREG):
            buf_v[pl.ds(z * 16, 16)] = neginf

        pltpu.make_async_copy(logits_hbm.at[row], row_v, dma_sem).wait()

        def mm_body(c, carry):
            accs = list(carry)
            for u in range(4):
                v = row_v[pl.ds((c * 4 + u) * 128, 16)]
                accs[u] = (jnp.maximum(accs[u][0], -v),
                           jnp.maximum(accs[u][1], v))
            return tuple(accs)
        mm = lax.fori_loop(0, _CHUNKS // 32, mm_body,
                           tuple((neginf, neginf) for _ in range(4)))
        nlo_v = jnp.maximum(jnp.maximum(mm[0][0], mm[1][0]),
                            jnp.maximum(mm[2][0], mm[3][0]))
        hi_v = jnp.maximum(jnp.maximum(mm[0][1], mm[1][1]),
                           jnp.maximum(mm[2][1], mm[3][1]))
        lo = -jnp.max(nlo_v)
        scale = (jnp.full((16,), _NBINS - 2, jnp.float32)
                 / jnp.broadcast_to(jnp.max(hi_v) - lo, (16,)))

        def to_bin(v):
            ti = ((v - lo) * scale).astype(jnp.int32)
            return jnp.clip(ti, 0, _NBINS - 1)

        def hist_body(w, _c):
            slots, wma, wmb = [], neginf, neginf
            for u in range(_WCHUNKS):
                v = row_v[pl.ds((w * _WCHUNKS + u) * 16, 16)]
                slots.append(lane_base + to_bin(v))
                if u % 2 == 0:
                    wma = jnp.maximum(wma, v)
                else:
                    wmb = jnp.maximum(wmb, v)
            for u in range(_WCHUNKS):
                h = hist if u % 2 == 0 else hist2
                plsc.addupdate_scatter(h, [slots[u]], ones_i)
            wmax[pl.ds(w * 16, 16)] = jnp.maximum(wma, wmb)
            return 0
        lax.fori_loop(0, _NWIN, hist_body, 0)

        def scan_body(i, carry):
            found, bin_b, c_gt, n_b, cum = carry
            cc = (_NBINS // 16 - 1) - i

            chunk = zeros_i
            for sh in range(16):
                chunk = (chunk + hist[pl.ds(sh * _NBINS + cc * 16, 16)]
                         + hist2[pl.ds(sh * _NBINS + cc * 16, 16)])
            rchunk = lax.rev(chunk, (0,))
            sfx = plsc.cumsum(rchunk) + cum
            ge = sfx >= _TOP_K
            cnt_ge = jnp.sum(ge.astype(jnp.int32))
            pos = jnp.int32(16) - cnt_ge
            s_pos = jnp.min(jnp.where(ge, sfx, jnp.int32(_BIG)))
            prev = jnp.maximum(jnp.max(jnp.where(ge, jnp.int32(-_BIG), sfx)),
                               cum)
            hit = jnp.logical_and(jnp.logical_not(found), cnt_ge > 0)
            return (jnp.logical_or(found, cnt_ge > 0),
                    jnp.where(hit, cc * 16 + 15 - pos, bin_b),
                    jnp.where(hit, prev, c_gt),
                    jnp.where(hit, s_pos - prev, n_b),
                    cum + jnp.sum(rchunk))
        _, bin_b, c_gt, n_b, _ = lax.fori_loop(
            0, _NBINS // 16, scan_body,
            (jnp.bool_(False), jnp.int32(0), jnp.int32(0), jnp.int32(0),
             jnp.int32(0)))

        lf_v = (lo + (jnp.broadcast_to(bin_b, (16,)).astype(jnp.float32)
                      - 2.0) / scale)

        def col_window(w, carry):
            hit = jnp.any(wmax[pl.ds(w * 16, 16)] >= lf_v)

            def do(carry):
                cnt_hi, cnt_in = carry
                for u in range(_WCHUNKS):
                    c = w * _WCHUNKS + u
                    v = row_v[pl.ds(c * 16, 16)]
                    binv = to_bin(v)
                    mhi = binv > bin_b
                    min_ = jnp.logical_and(binv == bin_b, cnt_in < _CAP_IN)
                    nhi = jnp.sum(mhi.astype(jnp.int32))
                    nin = jnp.sum(min_.astype(jnp.int32))

                    @pl.when(nhi + nin > 0)
                    def _store(cnt_hi=cnt_hi, cnt_in=cnt_in, v=v,
                               mhi=mhi, min_=min_, c=c):
                        idxv = c * 16 + iota
                        plsc.store_compressed(buf_v.at[pl.ds(cnt_hi, 16)],
                                              v, mask=mhi)
                        plsc.store_compressed(buf_i.at[pl.ds(cnt_hi, 16)],
                                              idxv, mask=mhi)
                        plsc.store_compressed(
                            buf_v.at[pl.ds(_HI_REGION + cnt_in, 16)],
                            v, mask=min_)
                        plsc.store_compressed(
                            buf_i.at[pl.ds(_HI_REGION + cnt_in, 16)],
                            idxv, mask=min_)
                    cnt_hi = cnt_hi + nhi
                    cnt_in = cnt_in + nin
                return (cnt_hi, cnt_in)

            return lax.cond(hit, do, lambda c_: c_, carry)
        lax.fori_loop(0, _NWIN, col_window, (jnp.int32(0), jnp.int32(0)))

        @pl.when(r < _ROWS_PER_W - 1)
        def _prefetch():
            pltpu.async_copy(logits_hbm.at[row + 1], row_v, dma_sem)

        def sel_body(i, _c):
            bv, bp = neginf, jnp.full((16,), _BIG, jnp.int32)
            for j in range(_NVREG):
                x = buf_v[pl.ds(j * 16, 16)]
                m = x > bv
                bv = jnp.where(m, x, bv)
                bp = jnp.where(m, j * 16 + iota, bp)
            mx = jnp.max(bv)
            p = jnp.min(jnp.where(bv == mx, bp, jnp.int32(_BIG)))
            pv = jnp.broadcast_to(p, (16,))
            idx_p = jnp.max(plsc.load_gather(buf_i, [pv]))
            plsc.store_scatter(buf_v, [pv], neginf, mask=lane0)
            iv = jnp.broadcast_to(i, (16,))
            plsc.store_scatter(out_v, [iv], jnp.broadcast_to(mx, (16,)),
                               mask=lane0)
            plsc.store_scatter(out_i, [iv], jnp.broadcast_to(idx_p, (16,)),
                               mask=lane0)
            return 0
        lax.fori_loop(0, _TOP_K, sel_body, 0)

        vals = [out_v[pl.ds(j * 16, 16)] for j in range(4)]
        vmax = jnp.max(vals[0])
        es = [jnp.exp(v - vmax) for v in vals]
        s = es[0] + es[1] + es[2] + es[3]
        total = jnp.sum(s)
        keeps, masked = [], []
        carry = jnp.float32(0.0)
        for j in range(4):
            pj = es[j] / total
            cj = plsc.cumsum(pj) + carry
            carry = jnp.max(cj)
            kp = cj <= _TOP_P
            if j == 0:
                kp = jnp.logical_or(kp, lane0)
            keeps.append(kp)
            masked.append(jnp.where(kp, vals[j], -jnp.inf)
                          + kinf_v[pl.ds(j * 16, 16)])
        bm, bp = neginf, jnp.full((16,), _BIG, jnp.int32)
        for j in range(4):
            sc = masked[j] + g_v[pl.ds(j * 16, 16)]
            m = sc > bm
            bm = jnp.where(m, sc, bm)
            bp = jnp.where(m, j * 16 + iota, bp)
        mx2 = jnp.max(bm)
        p2 = jnp.min(jnp.where(bm == mx2, bp, jnp.int32(_BIG)))
        token = jnp.max(plsc.load_gather(out_i, [jnp.broadcast_to(p2, (16,))]))

        e2 = [jnp.where(jnp.logical_and(
                  keeps[j], kinf_v[pl.ds(j * 16, 16)] == 0.0),
                  es[j], 0.0) for j in range(4)]
        s2 = jnp.sum(e2[0] + e2[1] + e2[2] + e2[3])
        for j in range(4):
            probs_st[pl.ds(j * 16, 16)] = e2[j] / s2
        tok_st[...] = jnp.where(lane0, token, 0)

        pltpu.sync_copy(probs_st, probs_hbm.at[row])
        pltpu.sync_copy(tok_st, tok_hbm.at[row])
        return 0

    lax.fori_loop(0, _ROWS_PER_W, row_body, 0)


@functools.partial(jax.jit, static_argnames=())
def _sc_topk_sample(logits, gumbel, kinf):
    mesh = plsc.VectorSubcoreMesh(core_axis_name="c", subcore_axis_name="s")
    f = pl.kernel(
        _sc_body,
        mesh=mesh,
        compiler_params=pltpu.CompilerParams(needs_layout_passes=False),
        out_type=(
            jax.ShapeDtypeStruct((_R, 16), jnp.int32),
            jax.ShapeDtypeStruct((_R, _TOP_K), jnp.float32),
        ),
        scratch_types=[
            pltpu.VMEM((_V,), jnp.float32),
            pltpu.VMEM((_HIST_SLOTS,), jnp.int32),
            pltpu.VMEM((_HIST_SLOTS,), jnp.int32),
            pltpu.VMEM((_NWIN * 16,), jnp.float32),
            pltpu.VMEM((_BUF,), jnp.float32),
            pltpu.VMEM((_BUF,), jnp.int32),
            pltpu.VMEM((_TOP_K,), jnp.float32),
            pltpu.VMEM((_TOP_K,), jnp.int32),
            pltpu.VMEM((_TOP_K,), jnp.float32),
            pltpu.VMEM((_TOP_K,), jnp.float32),
            pltpu.VMEM((_TOP_K,), jnp.float32),
            pltpu.VMEM((16,), jnp.int32),
            pltpu.SemaphoreType.DMA,
        ],
    )
    return f(logits, gumbel, kinf)


def kernel(logits, k):
    gumbel = jax.random.gumbel(jax.random.key(42), (_R, _TOP_K), jnp.float32)
    kinf = jnp.where(jnp.arange(_TOP_K) < k, 0.0, -jnp.inf).astype(jnp.float32)
    tok, probs = _sc_topk_sample(logits, gumbel, kinf)
    return tok[:, 0], probs


# speedup vs baseline: 1.8685x; 1.0030x over previous
import functools

import jax
import jax.numpy as jnp
from jax import lax
from jax.experimental import pallas as pl
from jax.experimental.pallas import tpu as pltpu
from jax.experimental.pallas import tpu_sc as plsc

_TOP_P = 0.9
_TOP_K = 64

_R = 128
_V = 100000
_NW = 32
_ROWS_PER_W = _R // _NW
_CHUNKS = _V // 16

_NBINS = 256
_HIST_SLOTS = 16 * _NBINS

_WCHUNKS = 10
_NWIN = _CHUNKS // _WCHUNKS

_HI_REGION = 96
_CAP_IN = 240
_BUF = _HI_REGION + _CAP_IN + 16
_NVREG = _BUF // 16

_BIG = 1 << 30


def _sc_body(logits_hbm, gumbel_hbm, kinf_hbm, tok_hbm, probs_hbm,
             row_v, hist, hist2, wmax, buf_v, buf_i, out_v, out_i, g_v,
             kinf_v, probs_st, tok_st, dma_sem):
    wid = lax.axis_index("s") * 2 + lax.axis_index("c")
    iota = lax.iota(jnp.int32, 16)
    lane0 = iota == 0
    zeros_i = jnp.zeros((16,), jnp.int32)
    ones_i = jnp.ones((16,), jnp.int32)
    neginf = jnp.full((16,), -jnp.inf, jnp.float32)
    lane_base = iota * jnp.int32(_NBINS)

    pltpu.sync_copy(kinf_hbm, kinf_v)
    first_row = wid * _ROWS_PER_W
    pltpu.async_copy(logits_hbm.at[first_row], row_v, dma_sem)

    def row_body(r, _):
        row = wid * _ROWS_PER_W + r
        pltpu.sync_copy(gumbel_hbm.at[row], g_v)

        def zh(z, _c):
            for u in range(8):
                hist[pl.ds((z * 8 + u) * 16, 16)] = zeros_i
                hist2[pl.ds((z * 8 + u) * 16, 16)] = zeros_i
            return 0
        lax.fori_loop(0, _HIST_SLOTS // 128, zh, 0)
        for z in range(_NVREG):
            buf_v[pl.ds(z * 16, 16)] = neginf

        pltpu.make_async_copy(logits_hbm.at[row], row_v, dma_sem).wait()

        def mm_body(c, carry):
            accs = list(carry)
            for u in range(4):
                v = row_v[pl.ds((c * 4 + u) * 128, 16)]
                accs[u] = (jnp.maximum(accs[u][0], -v),
                           jnp.maximum(accs[u][1], v))
            return tuple(accs)
        mm = lax.fori_loop(0, _CHUNKS // 32, mm_body,
                           tuple((neginf, neginf) for _ in range(4)))
        nlo_v = jnp.maximum(jnp.maximum(mm[0][0], mm[1][0]),
                            jnp.maximum(mm[2][0], mm[3][0]))
        hi_v = jnp.maximum(jnp.maximum(mm[0][1], mm[1][1]),
                           jnp.maximum(mm[2][1], mm[3][1]))
        lo = -jnp.max(nlo_v)
        scale = (jnp.full((16,), _NBINS - 2, jnp.float32)
                 / jnp.broadcast_to(jnp.max(hi_v) - lo, (16,)))

        def to_bin(v):
            ti = ((v - lo) * scale).astype(jnp.int32)
            return jnp.clip(ti, 0, _NBINS - 1)

        nls = lane_base.astype(jnp.float32) - lo * scale
        lane_hi = lane_base + jnp.int32(_NBINS - 1)

        def to_slot(v):
            ti = (v * scale + nls).astype(jnp.int32)
            return jnp.minimum(jnp.maximum(ti, lane_base), lane_hi)

        def hist_body(w, _c):
            slots, wma, wmb = [], neginf, neginf
            for u in range(_WCHUNKS):
                v = row_v[pl.ds((w * _WCHUNKS + u) * 16, 16)]
                slots.append(to_slot(v))
                if u % 2 == 0:
                    wma = jnp.maximum(wma, v)
                else:
                    wmb = jnp.maximum(wmb, v)
            for u in range(_WCHUNKS):
                h = hist if u % 2 == 0 else hist2
                plsc.addupdate_scatter(h, [slots[u]], ones_i)
            wmax[pl.ds(w * 16, 16)] = jnp.maximum(wma, wmb)
            return 0
        lax.fori_loop(0, _NWIN, hist_body, 0)

        def scan_body(i, carry):
            found, bin_b, c_gt, n_b, cum = carry
            cc = (_NBINS // 16 - 1) - i

            chunk = zeros_i
            for sh in range(16):
                chunk = (chunk + hist[pl.ds(sh * _NBINS + cc * 16, 16)]
                         + hist2[pl.ds(sh * _NBINS + cc * 16, 16)])
            rchunk = lax.rev(chunk, (0,))
            sfx = plsc.cumsum(rchunk) + cum
            ge = sfx >= _TOP_K
            cnt_ge = jnp.sum(ge.astype(jnp.int32))
            pos = jnp.int32(16) - cnt_ge
            s_pos = jnp.min(jnp.where(ge, sfx, jnp.int32(_BIG)))
            prev = jnp.maximum(jnp.max(jnp.where(ge, jnp.int32(-_BIG), sfx)),
                               cum)
            hit = jnp.logical_and(jnp.logical_not(found), cnt_ge > 0)
            return (jnp.logical_or(found, cnt_ge > 0),
                    jnp.where(hit, cc * 16 + 15 - pos, bin_b),
                    jnp.where(hit, prev, c_gt),
                    jnp.where(hit, s_pos - prev, n_b),
                    cum + jnp.sum(rchunk))
        _, bin_b, c_gt, n_b, _ = lax.fori_loop(
            0, _NBINS // 16, scan_body,
            (jnp.bool_(False), jnp.int32(0), jnp.int32(0), jnp.int32(0),
             jnp.int32(0)))

        lf_v = (lo + (jnp.broadcast_to(bin_b, (16,)).astype(jnp.float32)
                      - 2.0) / scale)
        thr_v = lane_base + jnp.broadcast_to(bin_b, (16,))

        def col_window(w, carry):
            hit = jnp.any(wmax[pl.ds(w * 16, 16)] >= lf_v)

            def do(carry):
                cnt_hi, cnt_in = carry
                for u in range(_WCHUNKS):
                    c = w * _WCHUNKS + u
                    v = row_v[pl.ds(c * 16, 16)]
                    slotv = to_slot(v)
                    mhi = slotv > thr_v
                    min_ = jnp.logical_and(slotv == thr_v, cnt_in < _CAP_IN)
                    nhi = jnp.sum(mhi.astype(jnp.int32))
                    nin = jnp.sum(min_.astype(jnp.int32))

                    @pl.when(nhi + nin > 0)
                    def _store(cnt_hi=cnt_hi, cnt_in=cnt_in, v=v,
                               mhi=mhi, min_=min_, c=c):
                        idxv = c * 16 + iota
                        plsc.store_compressed(buf_v.at[pl.ds(cnt_hi, 16)],
                                              v, mask=mhi)
                        plsc.store_compressed(buf_i.at[pl.ds(cnt_hi, 16)],
                                              idxv, mask=mhi)
                        plsc.store_compressed(
                            buf_v.at[pl.ds(_HI_REGION + cnt_in, 16)],
                            v, mask=min_)
                        plsc.store_compressed(
                            buf_i.at[pl.ds(_HI_REGION + cnt_in, 16)],
                            idxv, mask=min_)
                    cnt_hi = cnt_hi + nhi
                    cnt_in = cnt_in + nin
                return (cnt_hi, cnt_in)

            return lax.cond(hit, do, lambda c_: c_, carry)
        lax.fori_loop(0, _NWIN, col_window, (jnp.int32(0), jnp.int32(0)))

        @pl.when(r < _ROWS_PER_W - 1)
        def _prefetch():
            pltpu.async_copy(logits_hbm.at[row + 1], row_v, dma_sem)

        def sel_body(i, _c):
            bv, bp = neginf, jnp.full((16,), _BIG, jnp.int32)
            for j in range(_NVREG):
                x = buf_v[pl.ds(j * 16, 16)]
                m = x > bv
                bv = jnp.where(m, x, bv)
                bp = jnp.where(m, j * 16 + iota, bp)
            mx = jnp.max(bv)
            p = jnp.min(jnp.where(bv == mx, bp, jnp.int32(_BIG)))
            pv = jnp.broadcast_to(p, (16,))
            idx_p = jnp.max(plsc.load_gather(buf_i, [pv]))
            plsc.store_scatter(buf_v, [pv], neginf, mask=lane0)
            iv = jnp.broadcast_to(i, (16,))
            plsc.store_scatter(out_v, [iv], jnp.broadcast_to(mx, (16,)),
                               mask=lane0)
            plsc.store_scatter(out_i, [iv], jnp.broadcast_to(idx_p, (16,)),
                               mask=lane0)
            return 0
        lax.fori_loop(0, _TOP_K, sel_body, 0)

        vals = [out_v[pl.ds(j * 16, 16)] for j in range(4)]
        vmax = jnp.max(vals[0])
        es = [jnp.exp(v - vmax) for v in vals]
        s = es[0] + es[1] + es[2] + es[3]
        total = jnp.sum(s)
        keeps, masked = [], []
        carry = jnp.float32(0.0)
        for j in range(4):
            pj = es[j] / total
            cj = plsc.cumsum(pj) + carry
            carry = jnp.max(cj)
            kp = cj <= _TOP_P
            if j == 0:
                kp = jnp.logical_or(kp, lane0)
            keeps.append(kp)
            masked.append(jnp.where(kp, vals[j], -jnp.inf)
                          + kinf_v[pl.ds(j * 16, 16)])
        bm, bp = neginf, jnp.full((16,), _BIG, jnp.int32)
        for j in range(4):
            sc = masked[j] + g_v[pl.ds(j * 16, 16)]
            m = sc > bm
            bm = jnp.where(m, sc, bm)
            bp = jnp.where(m, j * 16 + iota, bp)
        mx2 = jnp.max(bm)
        p2 = jnp.min(jnp.where(bm == mx2, bp, jnp.int32(_BIG)))
        token = jnp.max(plsc.load_gather(out_i, [jnp.broadcast_to(p2, (16,))]))

        e2 = [jnp.where(jnp.logical_and(
                  keeps[j], kinf_v[pl.ds(j * 16, 16)] == 0.0),
                  es[j], 0.0) for j in range(4)]
        s2 = jnp.sum(e2[0] + e2[1] + e2[2] + e2[3])
        for j in range(4):
            probs_st[pl.ds(j * 16, 16)] = e2[j] / s2
        tok_st[...] = jnp.where(lane0, token, 0)

        pltpu.sync_copy(probs_st, probs_hbm.at[row])
        pltpu.sync_copy(tok_st, tok_hbm.at[row])
        return 0

    lax.fori_loop(0, _ROWS_PER_W, row_body, 0)


@functools.partial(jax.jit, static_argnames=())
def _sc_topk_sample(logits, gumbel, kinf):
    mesh = plsc.VectorSubcoreMesh(core_axis_name="c", subcore_axis_name="s")
    f = pl.kernel(
        _sc_body,
        mesh=mesh,
        compiler_params=pltpu.CompilerParams(needs_layout_passes=False),
        out_type=(
            jax.ShapeDtypeStruct((_R, 16), jnp.int32),
            jax.ShapeDtypeStruct((_R, _TOP_K), jnp.float32),
        ),
        scratch_types=[
            pltpu.VMEM((_V,), jnp.float32),
            pltpu.VMEM((_HIST_SLOTS,), jnp.int32),
            pltpu.VMEM((_HIST_SLOTS,), jnp.int32),
            pltpu.VMEM((_NWIN * 16,), jnp.float32),
            pltpu.VMEM((_BUF,), jnp.float32),
            pltpu.VMEM((_BUF,), jnp.int32),
            pltpu.VMEM((_TOP_K,), jnp.float32),
            pltpu.VMEM((_TOP_K,), jnp.int32),
            pltpu.VMEM((_TOP_K,), jnp.float32),
            pltpu.VMEM((_TOP_K,), jnp.float32),
            pltpu.VMEM((_TOP_K,), jnp.float32),
            pltpu.VMEM((16,), jnp.int32),
            pltpu.SemaphoreType.DMA,
        ],
    )
    return f(logits, gumbel, kinf)


def kernel(logits, k):
    gumbel = jax.random.gumbel(jax.random.key(42), (_R, _TOP_K), jnp.float32)
    kinf = jnp.where(jnp.arange(_TOP_K) < k, 0.0, -jnp.inf).astype(jnp.float32)
    tok, probs = _sc_topk_sample(logits, gumbel, kinf)
    return tok[:, 0], probs


# speedup vs baseline: 1.9855x; 1.0626x over previous
import functools

import jax
import jax.numpy as jnp
from jax import lax
from jax.experimental import pallas as pl
from jax.experimental.pallas import tpu as pltpu
from jax.experimental.pallas import tpu_sc as plsc

_TOP_P = 0.9
_TOP_K = 64

_R = 128
_V = 100000
_NW = 32
_ROWS_PER_W = _R // _NW
_CHUNKS = _V // 16

_NBINS = 256
_HIST_SLOTS = 16 * _NBINS

_WCHUNKS = 10
_NWIN = _CHUNKS // _WCHUNKS

_HI_REGION = 96
_CAP_IN = 240
_BUF = _HI_REGION + _CAP_IN + 16
_NVREG = _BUF // 16

_BIG = 1 << 30


def _sc_body(logits_hbm, gumbel_hbm, kinf_hbm, tok_hbm, probs_hbm,
             row_v, hist, hist2, wmax, buf_v, buf_i, out_v, out_i, g_v,
             kinf_v, probs_st, tok_st, dma_sem):
    wid = lax.axis_index("s") * 2 + lax.axis_index("c")
    iota = lax.iota(jnp.int32, 16)
    lane0 = iota == 0
    zeros_i = jnp.zeros((16,), jnp.int32)
    ones_i = jnp.ones((16,), jnp.int32)
    neginf = jnp.full((16,), -jnp.inf, jnp.float32)
    lane_base = iota * jnp.int32(_NBINS)

    pltpu.sync_copy(kinf_hbm, kinf_v)
    first_row = wid * _ROWS_PER_W
    pltpu.async_copy(logits_hbm.at[first_row], row_v, dma_sem)

    def row_body(r, _):
        row = wid * _ROWS_PER_W + r
        pltpu.sync_copy(gumbel_hbm.at[row], g_v)

        def zh(z, _c):
            for u in range(8):
                hist[pl.ds((z * 8 + u) * 16, 16)] = zeros_i
                hist2[pl.ds((z * 8 + u) * 16, 16)] = zeros_i
            return 0
        lax.fori_loop(0, _HIST_SLOTS // 128, zh, 0)
        for z in range(_NVREG):
            buf_v[pl.ds(z * 16, 16)] = neginf

        pltpu.make_async_copy(logits_hbm.at[row], row_v, dma_sem).wait()

        def mm_body(c, carry):
            accs = list(carry)
            for u in range(4):
                v = row_v[pl.ds((c * 4 + u) * 128, 16)]
                accs[u] = (jnp.maximum(accs[u][0], -v),
                           jnp.maximum(accs[u][1], v))
            return tuple(accs)
        mm = lax.fori_loop(0, _CHUNKS // 32, mm_body,
                           tuple((neginf, neginf) for _ in range(4)))
        nlo_v = jnp.maximum(jnp.maximum(mm[0][0], mm[1][0]),
                            jnp.maximum(mm[2][0], mm[3][0]))
        hi_v = jnp.maximum(jnp.maximum(mm[0][1], mm[1][1]),
                           jnp.maximum(mm[2][1], mm[3][1]))
        lo = -jnp.max(nlo_v)
        scale = (jnp.full((16,), _NBINS - 2, jnp.float32)
                 / jnp.broadcast_to(jnp.max(hi_v) - lo, (16,)))

        def to_bin(v):
            ti = ((v - lo) * scale).astype(jnp.int32)
            return jnp.clip(ti, 0, _NBINS - 1)

        nls = lane_base.astype(jnp.float32) - lo * scale
        lane_hi = lane_base + jnp.int32(_NBINS - 1)

        def to_slot(v):
            ti = (v * scale + nls).astype(jnp.int32)
            return jnp.minimum(jnp.maximum(ti, lane_base), lane_hi)

        def hist_body(w):
            slots, wma, wmb = [], neginf, neginf
            for u in range(_WCHUNKS):
                v = row_v[pl.ds((w * _WCHUNKS + u) * 16, 16)]
                slots.append(to_slot(v))
                if u % 2 == 0:
                    wma = jnp.maximum(wma, v)
                else:
                    wmb = jnp.maximum(wmb, v)
            for u in range(_WCHUNKS):
                h = hist if u % 2 == 0 else hist2
                plsc.addupdate_scatter(h, [slots[u]], ones_i)
            wmax[pl.ds(w * 16, 16)] = jnp.maximum(wma, wmb)
        plsc.parallel_loop(0, _NWIN, unroll=2)(hist_body)

        def scan_body(i, carry):
            found, bin_b, c_gt, n_b, cum = carry
            cc = (_NBINS // 16 - 1) - i

            chunk = zeros_i
            for sh in range(16):
                chunk = (chunk + hist[pl.ds(sh * _NBINS + cc * 16, 16)]
                         + hist2[pl.ds(sh * _NBINS + cc * 16, 16)])
            rchunk = lax.rev(chunk, (0,))
            sfx = plsc.cumsum(rchunk) + cum
            ge = sfx >= _TOP_K
            cnt_ge = jnp.sum(ge.astype(jnp.int32))
            pos = jnp.int32(16) - cnt_ge
            s_pos = jnp.min(jnp.where(ge, sfx, jnp.int32(_BIG)))
            prev = jnp.maximum(jnp.max(jnp.where(ge, jnp.int32(-_BIG), sfx)),
                               cum)
            hit = jnp.logical_and(jnp.logical_not(found), cnt_ge > 0)
            return (jnp.logical_or(found, cnt_ge > 0),
                    jnp.where(hit, cc * 16 + 15 - pos, bin_b),
                    jnp.where(hit, prev, c_gt),
                    jnp.where(hit, s_pos - prev, n_b),
                    cum + jnp.sum(rchunk))
        _, bin_b, c_gt, n_b, _ = lax.fori_loop(
            0, _NBINS // 16, scan_body,
            (jnp.bool_(False), jnp.int32(0), jnp.int32(0), jnp.int32(0),
             jnp.int32(0)))

        lf_v = (lo + (jnp.broadcast_to(bin_b, (16,)).astype(jnp.float32)
                      - 2.0) / scale)
        thr_v = lane_base + jnp.broadcast_to(bin_b, (16,))

        def col_window(w, carry):
            hit = jnp.any(wmax[pl.ds(w * 16, 16)] >= lf_v)

            def do(carry):
                cnt_hi, cnt_in = carry
                for u in range(_WCHUNKS):
                    c = w * _WCHUNKS + u
                    v = row_v[pl.ds(c * 16, 16)]
                    slotv = to_slot(v)
                    mhi = slotv > thr_v
                    min_ = jnp.logical_and(slotv == thr_v, cnt_in < _CAP_IN)
                    nhi = jnp.sum(mhi.astype(jnp.int32))
                    nin = jnp.sum(min_.astype(jnp.int32))

                    @pl.when(nhi + nin > 0)
                    def _store(cnt_hi=cnt_hi, cnt_in=cnt_in, v=v,
                               mhi=mhi, min_=min_, c=c):
                        idxv = c * 16 + iota
                        plsc.store_compressed(buf_v.at[pl.ds(cnt_hi, 16)],
                                              v, mask=mhi)
                        plsc.store_compressed(buf_i.at[pl.ds(cnt_hi, 16)],
                                              idxv, mask=mhi)
                        plsc.store_compressed(
                            buf_v.at[pl.ds(_HI_REGION + cnt_in, 16)],
                            v, mask=min_)
                        plsc.store_compressed(
                            buf_i.at[pl.ds(_HI_REGION + cnt_in, 16)],
                            idxv, mask=min_)
                    cnt_hi = cnt_hi + nhi
                    cnt_in = cnt_in + nin
                return (cnt_hi, cnt_in)

            return lax.cond(hit, do, lambda c_: c_, carry)
        lax.fori_loop(0, _NWIN, col_window, (jnp.int32(0), jnp.int32(0)))

        @pl.when(r < _ROWS_PER_W - 1)
        def _prefetch():
            pltpu.async_copy(logits_hbm.at[row + 1], row_v, dma_sem)

        def sel_body(i, _c):
            bv, bp = neginf, jnp.full((16,), _BIG, jnp.int32)
            for j in range(_NVREG):
                x = buf_v[pl.ds(j * 16, 16)]
                m = x > bv
                bv = jnp.where(m, x, bv)
                bp = jnp.where(m, j * 16 + iota, bp)
            mx = jnp.max(bv)
            p = jnp.min(jnp.where(bv == mx, bp, jnp.int32(_BIG)))
            pv = jnp.broadcast_to(p, (16,))
            idx_p = jnp.max(plsc.load_gather(buf_i, [pv]))
            plsc.store_scatter(buf_v, [pv], neginf, mask=lane0)
            iv = jnp.broadcast_to(i, (16,))
            plsc.store_scatter(out_v, [iv], jnp.broadcast_to(mx, (16,)),
                               mask=lane0)
            plsc.store_scatter(out_i, [iv], jnp.broadcast_to(idx_p, (16,)),
                               mask=lane0)
            return 0
        lax.fori_loop(0, _TOP_K, sel_body, 0)

        vals = [out_v[pl.ds(j * 16, 16)] for j in range(4)]
        vmax = jnp.max(vals[0])
        es = [jnp.exp(v - vmax) for v in vals]
        s = es[0] + es[1] + es[2] + es[3]
        total = jnp.sum(s)
        keeps, masked = [], []
        carry = jnp.float32(0.0)
        for j in range(4):
            pj = es[j] / total
            cj = plsc.cumsum(pj) + carry
            carry = jnp.max(cj)
            kp = cj <= _TOP_P
            if j == 0:
                kp = jnp.logical_or(kp, lane0)
            keeps.append(kp)
            masked.append(jnp.where(kp, vals[j], -jnp.inf)
                          + kinf_v[pl.ds(j * 16, 16)])
        bm, bp = neginf, jnp.full((16,), _BIG, jnp.int32)
        for j in range(4):
            sc = masked[j] + g_v[pl.ds(j * 16, 16)]
            m = sc > bm
            bm = jnp.where(m, sc, bm)
            bp = jnp.where(m, j * 16 + iota, bp)
        mx2 = jnp.max(bm)
        p2 = jnp.min(jnp.where(bm == mx2, bp, jnp.int32(_BIG)))
        token = jnp.max(plsc.load_gather(out_i, [jnp.broadcast_to(p2, (16,))]))

        e2 = [jnp.where(jnp.logical_and(
                  keeps[j], kinf_v[pl.ds(j * 16, 16)] == 0.0),
                  es[j], 0.0) for j in range(4)]
        s2 = jnp.sum(e2[0] + e2[1] + e2[2] + e2[3])
        for j in range(4):
            probs_st[pl.ds(j * 16, 16)] = e2[j] / s2
        tok_st[...] = jnp.where(lane0, token, 0)

        pltpu.sync_copy(probs_st, probs_hbm.at[row])
        pltpu.sync_copy(tok_st, tok_hbm.at[row])
        return 0

    lax.fori_loop(0, _ROWS_PER_W, row_body, 0)


@functools.partial(jax.jit, static_argnames=())
def _sc_topk_sample(logits, gumbel, kinf):
    mesh = plsc.VectorSubcoreMesh(core_axis_name="c", subcore_axis_name="s")
    f = pl.kernel(
        _sc_body,
        mesh=mesh,
        compiler_params=pltpu.CompilerParams(needs_layout_passes=False),
        out_type=(
            jax.ShapeDtypeStruct((_R, 16), jnp.int32),
            jax.ShapeDtypeStruct((_R, _TOP_K), jnp.float32),
        ),
        scratch_types=[
            pltpu.VMEM((_V,), jnp.float32),
            pltpu.VMEM((_HIST_SLOTS,), jnp.int32),
            pltpu.VMEM((_HIST_SLOTS,), jnp.int32),
            pltpu.VMEM((_NWIN * 16,), jnp.float32),
            pltpu.VMEM((_BUF,), jnp.float32),
            pltpu.VMEM((_BUF,), jnp.int32),
            pltpu.VMEM((_TOP_K,), jnp.float32),
            pltpu.VMEM((_TOP_K,), jnp.int32),
            pltpu.VMEM((_TOP_K,), jnp.float32),
            pltpu.VMEM((_TOP_K,), jnp.float32),
            pltpu.VMEM((_TOP_K,), jnp.float32),
            pltpu.VMEM((16,), jnp.int32),
            pltpu.SemaphoreType.DMA,
        ],
    )
    return f(logits, gumbel, kinf)


def kernel(logits, k):
    gumbel = jax.random.gumbel(jax.random.key(42), (_R, _TOP_K), jnp.float32)
    kinf = jnp.where(jnp.arange(_TOP_K) < k, 0.0, -jnp.inf).astype(jnp.float32)
    tok, probs = _sc_topk_sample(logits, gumbel, kinf)
    return tok[:, 0], probs
